# Initial kernel scaffold; baseline (speedup 1.0000x reference)
import jax
import jax.numpy as jnp
from jax.experimental import pallas as pl


def kernel(user_emb, item_emb, edge_weight, edge_index, users, items):
    raise NotImplementedError("write your pallas kernel here")


# speedup vs baseline: 12.0611x; 12.0611x over previous
import functools

import jax
import jax.numpy as jnp
from jax import lax
from jax.experimental import pallas as pl
from jax.experimental.pallas import tpu as pltpu
from jax.experimental.pallas import tpu_sc as plsc

D = 32
NUSERS = 20000
NITEMS = 30000
NNODES = NUSERS + NITEMS
NP = 51200
RPS = NP // 16
NC = 2
NS = 16
NW = NC * NS
CH = 512
GRP = 128
NG = CH // GRP
BATCH = 4096
BPW = BATCH // NW


def _scatter_body(emb_hbm, src_hbm, dst_hbm, w_hbm, zeros_hbm, part_hbm,
                  acc, src_v, dst_v, w_v, rows_v, gsem, *, nchunks):
  c = lax.axis_index("c")
  s = lax.axis_index("s")
  wid = c * NS + s
  iota = lax.iota(jnp.int32, 16)

  pltpu.sync_copy(zeros_hbm, acc.at[pl.ds(s * RPS, RPS)])
  plsc.subcore_barrier()

  def chunk_body(i, carry):
    row0 = wid * (nchunks * NG) + i * NG
    ebase = wid * (nchunks * CH) + i * CH
    pltpu.sync_copy(src_hbm.at[pl.ds(row0, NG)], src_v)
    pltpu.sync_copy(dst_hbm.at[pl.ds(row0, NG)], dst_v)
    pltpu.sync_copy(w_hbm.at[pl.ds(ebase, CH)], w_v)

    descs = []
    for g in range(NG):
      descs.append(pltpu.async_copy(
          emb_hbm.at[src_v.at[g]], rows_v.at[pl.ds(g * GRP, GRP)], gsem))
    for d in descs:
      d.wait()

    def wmul(j16, carry2):
      w16 = w_v[pl.ds(j16 * 16, 16)] * 0.5
      for j in range(16):
        e = j16 * 16 + j
        wb = lax.gather(
            w16, jnp.full((16, 1), j, jnp.int32),
            dimension_numbers=lax.GatherDimensionNumbers(
                offset_dims=(), collapsed_slice_dims=(0,),
                start_index_map=(0,)),
            slice_sizes=(1,),
            mode=lax.GatherScatterMode.PROMISE_IN_BOUNDS)
        r0 = rows_v[e, pl.ds(0, 16)]
        r1 = rows_v[e, pl.ds(16, 16)]
        rows_v[e, pl.ds(0, 16)] = r0 * wb
        rows_v[e, pl.ds(16, 16)] = r1 * wb
      return carry2

    lax.fori_loop(0, CH // 16, wmul, 0)

    for g in range(NG):
      pltpu.sync_copy(rows_v.at[pl.ds(g * GRP, GRP)],
                      acc.at[dst_v.at[g]], add=True)
    return carry

  lax.fori_loop(0, nchunks, chunk_body, 0)
  plsc.subcore_barrier()

  pltpu.sync_copy(acc.at[pl.ds(s * RPS, RPS)],
                  part_hbm.at[c, pl.ds(s * RPS, RPS)])


def _make_scatter(nchunks):
  mesh = plsc.VectorSubcoreMesh(core_axis_name="c", subcore_axis_name="s")
  return pl.kernel(
      functools.partial(_scatter_body, nchunks=nchunks),
      out_type=jax.ShapeDtypeStruct((NC, NP, D), jnp.float32),
      mesh=mesh,
      scratch_types=[
          pltpu.VMEM_SHARED((NP, D), jnp.float32),
          pltpu.VMEM((NG, GRP), jnp.int32),
          pltpu.VMEM((NG, GRP), jnp.int32),
          pltpu.VMEM((CH,), jnp.float32),
          pltpu.VMEM((CH, D), jnp.float32),
          pltpu.SemaphoreType.DMA,
      ],
      compiler_params=pltpu.CompilerParams(use_tc_tiling_on_sc=False, needs_layout_passes=False),
  )


def _combine_body(p0_ref, p1_ref, e_ref, t_ref, eo_ref, to_ref):
  v = 0.5 * e_ref[...] + p0_ref[...] + p1_ref[...]
  eo_ref[...] = v
  to_ref[...] = t_ref[...] + v


_COMBINE_BLK = 1024


def _combine(p0, p1, emb, tot):
  spec = pl.BlockSpec((_COMBINE_BLK, D), lambda i: (i, 0))
  return pl.pallas_call(
      _combine_body,
      grid=(NP // _COMBINE_BLK,),
      in_specs=[spec, spec, spec, spec],
      out_specs=[spec, spec],
      out_shape=[
          jax.ShapeDtypeStruct((NP, D), jnp.float32),
          jax.ShapeDtypeStruct((NP, D), jnp.float32),
      ],
  )(p0, p1, emb, tot)


def _readout_body(tot_hbm, un_hbm, in_hbm, out_hbm,
                  uv, iv, urows, irows, gv, sem):
  c = lax.axis_index("c")
  s = lax.axis_index("s")
  wid = c * NS + s
  base = wid * BPW
  iota = lax.iota(jnp.int32, 16)

  pltpu.sync_copy(un_hbm.at[pl.ds(base, BPW)], uv)
  pltpu.sync_copy(in_hbm.at[pl.ds(base, BPW)], iv)
  du = pltpu.async_copy(tot_hbm.at[uv], urows, sem)
  du.wait()
  di = pltpu.async_copy(tot_hbm.at[iv], irows, sem)
  di.wait()

  for q in range(BPW // 16):
    acc = jnp.zeros((16,), jnp.float32)
    for j in range(16):
      b = q * 16 + j
      p = (urows[b, pl.ds(0, 16)] * irows[b, pl.ds(0, 16)] +
           urows[b, pl.ds(16, 16)] * irows[b, pl.ds(16, 16)])
      acc = jnp.where(iota == j, jnp.sum(p), acc)
    gv[pl.ds(q * 16, 16)] = acc * 0.0625
  pltpu.sync_copy(gv, out_hbm.at[pl.ds(base, BPW)])


def _make_readout():
  mesh = plsc.VectorSubcoreMesh(core_axis_name="c", subcore_axis_name="s")
  return pl.kernel(
      _readout_body,
      out_type=jax.ShapeDtypeStruct((BATCH,), jnp.float32),
      mesh=mesh,
      scratch_types=[
          pltpu.VMEM((BPW,), jnp.int32),
          pltpu.VMEM((BPW,), jnp.int32),
          pltpu.VMEM((BPW, D), jnp.float32),
          pltpu.VMEM((BPW, D), jnp.float32),
          pltpu.VMEM((BPW,), jnp.float32),
          pltpu.SemaphoreType.DMA,
      ],
      compiler_params=pltpu.CompilerParams(use_tc_tiling_on_sc=False, needs_layout_passes=False),
  )


@jax.jit
def kernel(user_emb, item_emb, edge_weight, edge_index, users, items):
  e_total = edge_weight.shape[0]
  per_w = -(-e_total // (NW * CH)) * CH
  nchunks = per_w // CH
  e_pad = NW * per_w

  emb0 = jnp.concatenate([user_emb, item_emb], axis=0)
  emb0 = jnp.pad(emb0, ((0, NP - NNODES), (0, 0)))

  src = jnp.pad(edge_index[0], (0, e_pad - e_total))
  dst = jnp.pad(edge_index[1], (0, e_pad - e_total))
  w = jnp.pad(edge_weight, (0, e_pad - e_total))
  src2 = src.reshape(e_pad // GRP, GRP)
  dst2 = dst.reshape(e_pad // GRP, GRP)
  zeros = jnp.zeros((RPS, D), jnp.float32)

  scatter = _make_scatter(nchunks)
  emb = emb0
  tot = emb0
  for _ in range(3):
    part = scatter(emb, src2, dst2, w, zeros)
    emb, tot = _combine(part[0], part[1], emb, tot)

  readout = _make_readout()
  gamma = readout(tot, users, items + NUSERS)
  return gamma


# speedup vs baseline: 16.1859x; 1.3420x over previous
import functools

import jax
import jax.numpy as jnp
from jax import lax
from jax.experimental import pallas as pl
from jax.experimental.pallas import tpu as pltpu
from jax.experimental.pallas import tpu_sc as plsc

D = 32
NUSERS = 20000
NITEMS = 30000
NNODES = NUSERS + NITEMS
NP = 51200
RPS = NP // 16
NC = 2
NS = 16
NW = NC * NS
CH = 256
GRP = 128
NG = CH // GRP
BATCH = 4096
BPW = BATCH // NW


def _wmul(ebuf, rows):
  for gr in range(NG):
    for j16 in range(GRP // 16):
      w16 = plsc.bitcast(ebuf[2 * NG + gr, pl.ds(j16 * 16, 16)],
                         jnp.float32) * 0.5
      for j in range(16):
        e = gr * GRP + j16 * 16 + j
        wb = lax.gather(
            w16, jnp.full((16, 1), j, jnp.int32),
            dimension_numbers=lax.GatherDimensionNumbers(
                offset_dims=(), collapsed_slice_dims=(0,),
                start_index_map=(0,)),
            slice_sizes=(1,),
            mode=lax.GatherScatterMode.PROMISE_IN_BOUNDS)
        rows[e, pl.ds(0, 16)] = rows[e, pl.ds(0, 16)] * wb
        rows[e, pl.ds(16, 16)] = rows[e, pl.ds(16, 16)] * wb


def _scatter_body(emb_hbm, epk_hbm, zeros_hbm, part_hbm,
                  acc, ebuf0, ebuf1, rows0, rows1, gsem0, gsem1,
                  *, nchunks):
  c = lax.axis_index("c")
  s = lax.axis_index("s")
  wid = c * NS + s

  ebufs = (ebuf0, ebuf1)
  rowss = (rows0, rows1)
  gsems = (gsem0, gsem1)

  pltpu.sync_copy(zeros_hbm, acc.at[pl.ds(s * RPS, RPS)])
  plsc.subcore_barrier()

  def prefetch(i, b):
    pltpu.sync_copy(epk_hbm.at[wid * nchunks + i], ebufs[b])
    for g in range(NG):
      pltpu.async_copy(emb_hbm.at[ebufs[b].at[g]],
                       rowss[b].at[pl.ds(g * GRP, GRP)], gsems[b])

  prefetch(0, 0)
  prefetch(1, 1)

  def pair_body(i2, carry):
    for b in range(2):
      i = i2 * 2 + b
      pltpu.make_async_copy(emb_hbm.at[pl.ds(0, CH)], rowss[b],
                            gsems[b]).wait()
      _wmul(ebufs[b], rowss[b])
      for g in range(NG):
        pltpu.sync_copy(rowss[b].at[pl.ds(g * GRP, GRP)],
                        acc.at[ebufs[b].at[NG + g]], add=True)

      @pl.when(i + 2 < nchunks)
      def _():
        prefetch(i + 2, b)
    return carry

  lax.fori_loop(0, nchunks // 2, pair_body, 0)
  plsc.subcore_barrier()

  pltpu.sync_copy(acc.at[pl.ds(s * RPS, RPS)],
                  part_hbm.at[c, pl.ds(s * RPS, RPS)])


def _make_scatter(nchunks):
  mesh = plsc.VectorSubcoreMesh(core_axis_name="c", subcore_axis_name="s")
  return pl.kernel(
      functools.partial(_scatter_body, nchunks=nchunks),
      out_type=jax.ShapeDtypeStruct((NC, NP, D), jnp.float32),
      mesh=mesh,
      scratch_types=[
          pltpu.VMEM_SHARED((NP, D), jnp.float32),
          pltpu.VMEM((3 * NG, GRP), jnp.int32),
          pltpu.VMEM((3 * NG, GRP), jnp.int32),
          pltpu.VMEM((CH, D), jnp.float32),
          pltpu.VMEM((CH, D), jnp.float32),
          pltpu.SemaphoreType.DMA,
          pltpu.SemaphoreType.DMA,
      ],
      compiler_params=pltpu.CompilerParams(use_tc_tiling_on_sc=False, needs_layout_passes=False),
  )


def _combine_body(p0_ref, p1_ref, e_ref, t_ref, eo_ref, to_ref):
  v = 0.5 * e_ref[...] + p0_ref[...] + p1_ref[...]
  eo_ref[...] = v
  to_ref[...] = t_ref[...] + v


_COMBINE_BLK = 1024


def _combine(p0, p1, emb, tot):
  spec = pl.BlockSpec((_COMBINE_BLK, D), lambda i: (i, 0))
  return pl.pallas_call(
      _combine_body,
      grid=(NP // _COMBINE_BLK,),
      in_specs=[spec, spec, spec, spec],
      out_specs=[spec, spec],
      out_shape=[
          jax.ShapeDtypeStruct((NP, D), jnp.float32),
          jax.ShapeDtypeStruct((NP, D), jnp.float32),
      ],
  )(p0, p1, emb, tot)


def _readout_body(tot_hbm, un_hbm, in_hbm, out_hbm,
                  uv, iv, urows, irows, gv, sem):
  c = lax.axis_index("c")
  s = lax.axis_index("s")
  wid = c * NS + s
  base = wid * BPW
  iota = lax.iota(jnp.int32, 16)

  pltpu.sync_copy(un_hbm.at[pl.ds(base, BPW)], uv)
  pltpu.sync_copy(in_hbm.at[pl.ds(base, BPW)], iv)
  du = pltpu.async_copy(tot_hbm.at[uv], urows, sem)
  du.wait()
  di = pltpu.async_copy(tot_hbm.at[iv], irows, sem)
  di.wait()

  for q in range(BPW // 16):
    acc = jnp.zeros((16,), jnp.float32)
    for j in range(16):
      b = q * 16 + j
      p = (urows[b, pl.ds(0, 16)] * irows[b, pl.ds(0, 16)] +
           urows[b, pl.ds(16, 16)] * irows[b, pl.ds(16, 16)])
      acc = jnp.where(iota == j, jnp.sum(p), acc)
    gv[pl.ds(q * 16, 16)] = acc * 0.0625
  pltpu.sync_copy(gv, out_hbm.at[pl.ds(base, BPW)])


def _make_readout():
  mesh = plsc.VectorSubcoreMesh(core_axis_name="c", subcore_axis_name="s")
  return pl.kernel(
      _readout_body,
      out_type=jax.ShapeDtypeStruct((BATCH,), jnp.float32),
      mesh=mesh,
      scratch_types=[
          pltpu.VMEM((BPW,), jnp.int32),
          pltpu.VMEM((BPW,), jnp.int32),
          pltpu.VMEM((BPW, D), jnp.float32),
          pltpu.VMEM((BPW, D), jnp.float32),
          pltpu.VMEM((BPW,), jnp.float32),
          pltpu.SemaphoreType.DMA,
      ],
      compiler_params=pltpu.CompilerParams(use_tc_tiling_on_sc=False, needs_layout_passes=False),
  )


@jax.jit
def kernel(user_emb, item_emb, edge_weight, edge_index, users, items):
  e_total = edge_weight.shape[0]
  per_w = -(-e_total // (NW * 2 * CH)) * 2 * CH
  nchunks = per_w // CH
  e_pad = NW * per_w

  emb0 = jnp.concatenate([user_emb, item_emb], axis=0)
  emb0 = jnp.pad(emb0, ((0, NP - NNODES), (0, 0)))

  src = jnp.pad(edge_index[0], (0, e_pad - e_total))
  dst = jnp.pad(edge_index[1], (0, e_pad - e_total))
  w = jnp.pad(edge_weight, (0, e_pad - e_total))
  epk = jnp.concatenate([
      src.reshape(-1, NG, GRP),
      dst.reshape(-1, NG, GRP),
      lax.bitcast_convert_type(w, jnp.int32).reshape(-1, NG, GRP),
  ], axis=1)
  zeros = jnp.zeros((RPS, D), jnp.float32)

  scatter = _make_scatter(nchunks)
  emb = emb0
  tot = emb0
  for _ in range(3):
    part = scatter(emb, epk, zeros)
    emb, tot = _combine(part[0], part[1], emb, tot)

  readout = _make_readout()
  gamma = readout(tot, users, items + NUSERS)
  return gamma


# speedup vs baseline: 18.6968x; 1.1551x over previous
import functools

import jax
import jax.numpy as jnp
from jax import lax
from jax.experimental import pallas as pl
from jax.experimental.pallas import tpu as pltpu
from jax.experimental.pallas import tpu_sc as plsc

D = 32
NUSERS = 20000
NITEMS = 30000
NNODES = NUSERS + NITEMS
NP = 51200
RPS = NP // 16
NC = 2
NS = 16
NW = NC * NS
CH = 256
GRP = 128
NG = CH // GRP
BATCH = 4096
BPW = BATCH // NW


def _wmul(ebuf, rows):
  for gr in range(NG):
    for j16 in range(GRP // 16):
      w16 = plsc.bitcast(ebuf[2 * NG + gr, pl.ds(j16 * 16, 16)],
                         jnp.float32) * 0.5
      for j in range(16):
        e = gr * GRP + j16 * 16 + j
        wb = lax.gather(
            w16, jnp.full((16, 1), j, jnp.int32),
            dimension_numbers=lax.GatherDimensionNumbers(
                offset_dims=(), collapsed_slice_dims=(0,),
                start_index_map=(0,)),
            slice_sizes=(1,),
            mode=lax.GatherScatterMode.PROMISE_IN_BOUNDS)
        rows[e, pl.ds(0, 16)] = rows[e, pl.ds(0, 16)] * wb
        rows[e, pl.ds(16, 16)] = rows[e, pl.ds(16, 16)] * wb


MB = 80


def _merge_strips(part_hbm, pemb_hbm, ptot_hbm, emb_out, tot_out,
                  mb0, mb1, mb2, mb3, msem, s, *, write_emb):
  def strip(k, carry):
    r0 = s * RPS + k * MB
    d0 = pltpu.async_copy(part_hbm.at[0, pl.ds(r0, MB)], mb0, msem)
    d1 = pltpu.async_copy(part_hbm.at[1, pl.ds(r0, MB)], mb1, msem)
    d2 = pltpu.async_copy(pemb_hbm.at[pl.ds(r0, MB)], mb2, msem)
    d3 = pltpu.async_copy(ptot_hbm.at[pl.ds(r0, MB)], mb3, msem)
    d0.wait(); d1.wait(); d2.wait(); d3.wait()

    def rowfix(r, carry2):
      for h in range(2):
        sl = pl.ds(h * 16, 16)
        v = 0.5 * mb2[r, sl] + mb0[r, sl] + mb1[r, sl]
        mb0[r, sl] = v
        mb3[r, sl] = mb3[r, sl] + v
      return carry2

    lax.fori_loop(0, MB, rowfix, 0)
    if write_emb:
      pltpu.sync_copy(mb0, emb_out.at[pl.ds(r0, MB)])
    pltpu.sync_copy(mb3, tot_out.at[pl.ds(r0, MB)])
    return carry

  lax.fori_loop(0, RPS // MB, strip, 0)


def _scatter_phase2(emb_hbm, epk_hbm, part_hbm, acc,
                    ebufs, rowss, gsems, wid, s, c, nchunks):
  def prefetch(i, b):
    pltpu.sync_copy(epk_hbm.at[wid * nchunks + i], ebufs[b])
    for g in range(NG):
      pltpu.async_copy(emb_hbm.at[ebufs[b].at[g]],
                       rowss[b].at[pl.ds(g * GRP, GRP)], gsems[b])

  prefetch(0, 0)
  prefetch(1, 1)

  def pair_body(i2, carry):
    for b in range(2):
      i = i2 * 2 + b
      pltpu.make_async_copy(emb_hbm.at[pl.ds(0, CH)], rowss[b],
                            gsems[b]).wait()
      _wmul(ebufs[b], rowss[b])
      for g in range(NG):
        pltpu.sync_copy(rowss[b].at[pl.ds(g * GRP, GRP)],
                        acc.at[ebufs[b].at[NG + g]], add=True)

      @pl.when(i + 2 < nchunks)
      def _():
        prefetch(i + 2, b)
    return carry

  lax.fori_loop(0, nchunks // 2, pair_body, 0)
  plsc.subcore_barrier()

  pltpu.sync_copy(acc.at[pl.ds(s * RPS, RPS)],
                  part_hbm.at[c, pl.ds(s * RPS, RPS)])


def _scatter1_body(emb_hbm, epk_hbm, zeros_hbm, part_hbm,
                   acc, ebuf0, ebuf1, rows0, rows1, gsem0, gsem1,
                   *, nchunks):
  c = lax.axis_index("c")
  s = lax.axis_index("s")
  wid = c * NS + s
  pltpu.sync_copy(zeros_hbm, acc.at[pl.ds(s * RPS, RPS)])
  plsc.subcore_barrier()
  _scatter_phase2(emb_hbm, epk_hbm, part_hbm, acc,
                  (ebuf0, ebuf1), (rows0, rows1), (gsem0, gsem1),
                  wid, s, c, nchunks)


def _merge_scatter_body(pemb_hbm, ptot_hbm, prev_hbm, epk_hbm, zeros_hbm,
                        part_hbm, emb_hbm, tot_hbm,
                        acc, ebuf0, ebuf1, rows0, rows1,
                        mb0, mb1, mb2, mb3, gsem0, gsem1, msem,
                        *, nchunks):
  c = lax.axis_index("c")
  s = lax.axis_index("s")
  wid = c * NS + s
  pltpu.sync_copy(zeros_hbm, acc.at[pl.ds(s * RPS, RPS)])
  _merge_strips(prev_hbm, pemb_hbm, ptot_hbm, emb_hbm, tot_hbm,
                mb0, mb1, mb2, mb3, msem, s, write_emb=True)
  plsc.subcore_barrier()
  _scatter_phase2(emb_hbm, epk_hbm, part_hbm, acc,
                  (ebuf0, ebuf1), (rows0, rows1), (gsem0, gsem1),
                  wid, s, c, nchunks)


_SCATTER_SCRATCH = [
    pltpu.VMEM_SHARED((NP, D), jnp.float32),
    pltpu.VMEM((3 * NG, GRP), jnp.int32),
    pltpu.VMEM((3 * NG, GRP), jnp.int32),
    pltpu.VMEM((CH, D), jnp.float32),
    pltpu.VMEM((CH, D), jnp.float32),
]
_MERGE_SCRATCH = [
    pltpu.VMEM((MB, D), jnp.float32),
    pltpu.VMEM((MB, D), jnp.float32),
    pltpu.VMEM((MB, D), jnp.float32),
    pltpu.VMEM((MB, D), jnp.float32),
]
_SC_PARAMS = pltpu.CompilerParams(use_tc_tiling_on_sc=False,
                                  needs_layout_passes=False)


def _make_scatter1(nchunks):
  mesh = plsc.VectorSubcoreMesh(core_axis_name="c", subcore_axis_name="s")
  return pl.kernel(
      functools.partial(_scatter1_body, nchunks=nchunks),
      out_type=jax.ShapeDtypeStruct((NC, NP, D), jnp.float32),
      mesh=mesh,
      scratch_types=_SCATTER_SCRATCH + [
          pltpu.SemaphoreType.DMA, pltpu.SemaphoreType.DMA],
      compiler_params=_SC_PARAMS,
  )


def _make_merge_scatter(nchunks):
  mesh = plsc.VectorSubcoreMesh(core_axis_name="c", subcore_axis_name="s")
  return pl.kernel(
      functools.partial(_merge_scatter_body, nchunks=nchunks),
      out_type=(jax.ShapeDtypeStruct((NC, NP, D), jnp.float32),
                jax.ShapeDtypeStruct((NP, D), jnp.float32),
                jax.ShapeDtypeStruct((NP, D), jnp.float32)),
      mesh=mesh,
      scratch_types=_SCATTER_SCRATCH + _MERGE_SCRATCH + [
          pltpu.SemaphoreType.DMA, pltpu.SemaphoreType.DMA,
          pltpu.SemaphoreType.DMA],
      compiler_params=_SC_PARAMS,
  )


def _readout_body(pemb_hbm, ptot_hbm, prev_hbm, un_hbm, in_hbm,
                  out_hbm, tot_hbm,
                  uv, iv, urows, irows, gv, mb0, mb1, mb2, mb3, sem, msem):
  c = lax.axis_index("c")
  s = lax.axis_index("s")
  wid = c * NS + s
  base = wid * BPW
  iota = lax.iota(jnp.int32, 16)

  _merge_strips(prev_hbm, pemb_hbm, ptot_hbm, None, tot_hbm,
                mb0, mb1, mb2, mb3, msem, s, write_emb=False)
  plsc.subcore_barrier()

  pltpu.sync_copy(un_hbm.at[pl.ds(base, BPW)], uv)
  pltpu.sync_copy(in_hbm.at[pl.ds(base, BPW)], iv)
  du = pltpu.async_copy(tot_hbm.at[uv], urows, sem)
  du.wait()
  di = pltpu.async_copy(tot_hbm.at[iv], irows, sem)
  di.wait()

  for q in range(BPW // 16):
    acc = jnp.zeros((16,), jnp.float32)
    for j in range(16):
      b = q * 16 + j
      p = (urows[b, pl.ds(0, 16)] * irows[b, pl.ds(0, 16)] +
           urows[b, pl.ds(16, 16)] * irows[b, pl.ds(16, 16)])
      acc = jnp.where(iota == j, jnp.sum(p), acc)
    gv[pl.ds(q * 16, 16)] = acc * 0.0625
  pltpu.sync_copy(gv, out_hbm.at[pl.ds(base, BPW)])


def _make_readout():
  mesh = plsc.VectorSubcoreMesh(core_axis_name="c", subcore_axis_name="s")
  return pl.kernel(
      _readout_body,
      out_type=(jax.ShapeDtypeStruct((BATCH,), jnp.float32),
                jax.ShapeDtypeStruct((NP, D), jnp.float32)),
      mesh=mesh,
      scratch_types=[
          pltpu.VMEM((BPW,), jnp.int32),
          pltpu.VMEM((BPW,), jnp.int32),
          pltpu.VMEM((BPW, D), jnp.float32),
          pltpu.VMEM((BPW, D), jnp.float32),
          pltpu.VMEM((BPW,), jnp.float32),
      ] + _MERGE_SCRATCH + [
          pltpu.SemaphoreType.DMA,
          pltpu.SemaphoreType.DMA,
      ],
      compiler_params=_SC_PARAMS,
  )


@jax.jit
def kernel(user_emb, item_emb, edge_weight, edge_index, users, items):
  e_total = edge_weight.shape[0]
  per_w = -(-e_total // (NW * 2 * CH)) * 2 * CH
  nchunks = per_w // CH
  e_pad = NW * per_w

  emb0 = jnp.concatenate([user_emb, item_emb], axis=0)
  emb0 = jnp.pad(emb0, ((0, NP - NNODES), (0, 0)))

  src = jnp.pad(edge_index[0], (0, e_pad - e_total))
  dst = jnp.pad(edge_index[1], (0, e_pad - e_total))
  w = jnp.pad(edge_weight, (0, e_pad - e_total))
  epk = jnp.concatenate([
      src.reshape(-1, NG, GRP),
      dst.reshape(-1, NG, GRP),
      lax.bitcast_convert_type(w, jnp.int32).reshape(-1, NG, GRP),
  ], axis=1)
  zeros = jnp.zeros((RPS, D), jnp.float32)

  part1 = _make_scatter1(nchunks)(emb0, epk, zeros)
  mscat = _make_merge_scatter(nchunks)
  part2, emb1, tot1 = mscat(emb0, emb0, part1, epk, zeros)
  part3, emb2, tot2 = mscat(emb1, tot1, part2, epk, zeros)
  gamma, _ = _make_readout()(emb2, tot2, part3, users, items + NUSERS)
  return gamma


# speedup vs baseline: 19.3869x; 1.0369x over previous
import functools

import jax
import jax.numpy as jnp
from jax import lax
from jax.experimental import pallas as pl
from jax.experimental.pallas import tpu as pltpu
from jax.experimental.pallas import tpu_sc as plsc

D = 32
NUSERS = 20000
NITEMS = 30000
NNODES = NUSERS + NITEMS
NP = 51200
RPS = NP // 16
NC = 2
NS = 16
NW = NC * NS
CH = 256
GRP = 128
NG = CH // GRP
BATCH = 4096
BPW = BATCH // NW


def _wmul(ebuf, rows):
  for gr in range(NG):
    for j16 in range(GRP // 16):
      w16 = plsc.bitcast(ebuf[2 * NG + gr, pl.ds(j16 * 16, 16)],
                         jnp.float32) * 0.5
      for j in range(16):
        e = gr * GRP + j16 * 16 + j
        wb = lax.gather(
            w16, jnp.full((16, 1), j, jnp.int32),
            dimension_numbers=lax.GatherDimensionNumbers(
                offset_dims=(), collapsed_slice_dims=(0,),
                start_index_map=(0,)),
            slice_sizes=(1,),
            mode=lax.GatherScatterMode.PROMISE_IN_BOUNDS)
        rows[e, pl.ds(0, 16)] = rows[e, pl.ds(0, 16)] * wb
        rows[e, pl.ds(16, 16)] = rows[e, pl.ds(16, 16)] * wb


MB = 64


def _merge_strips(part_hbm, pemb_hbm, ptot_hbm, emb_out, tot_out,
                  mb0, mb1, mb2, mb3, msem, s, *, write_emb):
  def strip(k, carry):
    r0 = s * RPS + k * MB
    d0 = pltpu.async_copy(part_hbm.at[0, pl.ds(r0, MB)], mb0, msem)
    d1 = pltpu.async_copy(part_hbm.at[1, pl.ds(r0, MB)], mb1, msem)
    d2 = pltpu.async_copy(pemb_hbm.at[pl.ds(r0, MB)], mb2, msem)
    d3 = pltpu.async_copy(ptot_hbm.at[pl.ds(r0, MB)], mb3, msem)
    d0.wait(); d1.wait(); d2.wait(); d3.wait()

    def rowfix(r, carry2):
      for h in range(2):
        sl = pl.ds(h * 16, 16)
        v = 0.5 * mb2[r, sl] + mb0[r, sl] + mb1[r, sl]
        mb0[r, sl] = v
        mb3[r, sl] = mb3[r, sl] + v
      return carry2

    lax.fori_loop(0, MB, rowfix, 0)
    if write_emb:
      pltpu.sync_copy(mb0, emb_out.at[pl.ds(r0, MB)])
    pltpu.sync_copy(mb3, tot_out.at[pl.ds(r0, MB)])
    return carry

  lax.fori_loop(0, RPS // MB, strip, 0)


def _scatter_phase2(emb_hbm, epk_hbm, part_hbm, acc,
                    ebufs, rowss, gsems, esems, wid, s, c, nchunks):
  def chunk_id(i):
    return i * NW + wid

  def gather_issue(i, slot, b):
    for g in range(NG):
      pltpu.async_copy(emb_hbm.at[ebufs[slot].at[g]],
                       rowss[b].at[pl.ds(g * GRP, GRP)], gsems[b])

  for i in range(2):
    pltpu.sync_copy(epk_hbm.at[chunk_id(i)], ebufs[i])
    gather_issue(i, i, i)

  def quad_body(i4, carry):
    for q in range(4):
      i = i4 * 4 + q
      b = q % 2
      slot2 = (q + 2) % 4

      @pl.when(i + 2 < nchunks)
      def _():
        pltpu.async_copy(epk_hbm.at[chunk_id(i + 2)], ebufs[slot2],
                         esems[b])

      pltpu.make_async_copy(emb_hbm.at[pl.ds(0, CH)], rowss[b],
                            gsems[b]).wait()
      _wmul(ebufs[q], rowss[b])
      for g in range(NG):
        pltpu.sync_copy(rowss[b].at[pl.ds(g * GRP, GRP)],
                        acc.at[ebufs[q].at[NG + g]], add=True)

      @pl.when(i + 2 < nchunks)
      def _():
        pltpu.make_async_copy(epk_hbm.at[0], ebufs[slot2],
                              esems[b]).wait()
        gather_issue(i + 2, slot2, b)
    return carry

  lax.fori_loop(0, nchunks // 4, quad_body, 0)
  plsc.subcore_barrier()

  pltpu.sync_copy(acc.at[pl.ds(s * RPS, RPS)],
                  part_hbm.at[c, pl.ds(s * RPS, RPS)])


def _scatter1_body(emb_hbm, epk_hbm, zeros_hbm, part_hbm,
                   acc, ebuf0, ebuf1, ebuf2, ebuf3, rows0, rows1,
                   gsem0, gsem1, esem0, esem1,
                   *, nchunks):
  c = lax.axis_index("c")
  s = lax.axis_index("s")
  wid = c * NS + s
  pltpu.sync_copy(zeros_hbm, acc.at[pl.ds(s * RPS, RPS)])
  plsc.subcore_barrier()
  _scatter_phase2(emb_hbm, epk_hbm, part_hbm, acc,
                  (ebuf0, ebuf1, ebuf2, ebuf3), (rows0, rows1),
                  (gsem0, gsem1), (esem0, esem1),
                  wid, s, c, nchunks)


def _merge_scatter_body(pemb_hbm, ptot_hbm, prev_hbm, epk_hbm, zeros_hbm,
                        part_hbm, emb_hbm, tot_hbm,
                        acc, ebuf0, ebuf1, ebuf2, ebuf3, rows0, rows1,
                        mb0, mb1, mb2, mb3,
                        gsem0, gsem1, esem0, esem1, msem,
                        *, nchunks):
  c = lax.axis_index("c")
  s = lax.axis_index("s")
  wid = c * NS + s
  pltpu.sync_copy(zeros_hbm, acc.at[pl.ds(s * RPS, RPS)])
  _merge_strips(prev_hbm, pemb_hbm, ptot_hbm, emb_hbm, tot_hbm,
                mb0, mb1, mb2, mb3, msem, s, write_emb=True)
  plsc.subcore_barrier()
  _scatter_phase2(emb_hbm, epk_hbm, part_hbm, acc,
                  (ebuf0, ebuf1, ebuf2, ebuf3), (rows0, rows1),
                  (gsem0, gsem1), (esem0, esem1),
                  wid, s, c, nchunks)


_SCATTER_SCRATCH = [
    pltpu.VMEM_SHARED((NP, D), jnp.float32),
    pltpu.VMEM((3 * NG, GRP), jnp.int32),
    pltpu.VMEM((3 * NG, GRP), jnp.int32),
    pltpu.VMEM((3 * NG, GRP), jnp.int32),
    pltpu.VMEM((3 * NG, GRP), jnp.int32),
    pltpu.VMEM((CH, D), jnp.float32),
    pltpu.VMEM((CH, D), jnp.float32),
]
_MERGE_SCRATCH = [
    pltpu.VMEM((MB, D), jnp.float32),
    pltpu.VMEM((MB, D), jnp.float32),
    pltpu.VMEM((MB, D), jnp.float32),
    pltpu.VMEM((MB, D), jnp.float32),
]
_SC_PARAMS = pltpu.CompilerParams(use_tc_tiling_on_sc=False,
                                  needs_layout_passes=False)


def _make_scatter1(nchunks):
  mesh = plsc.VectorSubcoreMesh(core_axis_name="c", subcore_axis_name="s")
  return pl.kernel(
      functools.partial(_scatter1_body, nchunks=nchunks),
      out_type=jax.ShapeDtypeStruct((NC, NP, D), jnp.float32),
      mesh=mesh,
      scratch_types=_SCATTER_SCRATCH + [
          pltpu.SemaphoreType.DMA, pltpu.SemaphoreType.DMA,
          pltpu.SemaphoreType.DMA, pltpu.SemaphoreType.DMA],
      compiler_params=_SC_PARAMS,
  )


def _make_merge_scatter(nchunks):
  mesh = plsc.VectorSubcoreMesh(core_axis_name="c", subcore_axis_name="s")
  return pl.kernel(
      functools.partial(_merge_scatter_body, nchunks=nchunks),
      out_type=(jax.ShapeDtypeStruct((NC, NP, D), jnp.float32),
                jax.ShapeDtypeStruct((NP, D), jnp.float32),
                jax.ShapeDtypeStruct((NP, D), jnp.float32)),
      mesh=mesh,
      scratch_types=_SCATTER_SCRATCH + _MERGE_SCRATCH + [
          pltpu.SemaphoreType.DMA, pltpu.SemaphoreType.DMA,
          pltpu.SemaphoreType.DMA, pltpu.SemaphoreType.DMA,
          pltpu.SemaphoreType.DMA],
      compiler_params=_SC_PARAMS,
  )


def _readout_body(pemb_hbm, ptot_hbm, prev_hbm, un_hbm, in_hbm,
                  out_hbm, tot_hbm,
                  uv, iv, urows, irows, gv, mb0, mb1, mb2, mb3, sem, msem):
  c = lax.axis_index("c")
  s = lax.axis_index("s")
  wid = c * NS + s
  base = wid * BPW
  iota = lax.iota(jnp.int32, 16)

  _merge_strips(prev_hbm, pemb_hbm, ptot_hbm, None, tot_hbm,
                mb0, mb1, mb2, mb3, msem, s, write_emb=False)
  plsc.subcore_barrier()

  pltpu.sync_copy(un_hbm.at[pl.ds(base, BPW)], uv)
  pltpu.sync_copy(in_hbm.at[pl.ds(base, BPW)], iv)
  du = pltpu.async_copy(tot_hbm.at[uv], urows, sem)
  du.wait()
  di = pltpu.async_copy(tot_hbm.at[iv], irows, sem)
  di.wait()

  for q in range(BPW // 16):
    acc = jnp.zeros((16,), jnp.float32)
    for j in range(16):
      b = q * 16 + j
      p = (urows[b, pl.ds(0, 16)] * irows[b, pl.ds(0, 16)] +
           urows[b, pl.ds(16, 16)] * irows[b, pl.ds(16, 16)])
      acc = jnp.where(iota == j, jnp.sum(p), acc)
    gv[pl.ds(q * 16, 16)] = acc * 0.0625
  pltpu.sync_copy(gv, out_hbm.at[pl.ds(base, BPW)])


def _make_readout():
  mesh = plsc.VectorSubcoreMesh(core_axis_name="c", subcore_axis_name="s")
  return pl.kernel(
      _readout_body,
      out_type=(jax.ShapeDtypeStruct((BATCH,), jnp.float32),
                jax.ShapeDtypeStruct((NP, D), jnp.float32)),
      mesh=mesh,
      scratch_types=[
          pltpu.VMEM((BPW,), jnp.int32),
          pltpu.VMEM((BPW,), jnp.int32),
          pltpu.VMEM((BPW, D), jnp.float32),
          pltpu.VMEM((BPW, D), jnp.float32),
          pltpu.VMEM((BPW,), jnp.float32),
      ] + _MERGE_SCRATCH + [
          pltpu.SemaphoreType.DMA,
          pltpu.SemaphoreType.DMA,
      ],
      compiler_params=_SC_PARAMS,
  )


@jax.jit
def kernel(user_emb, item_emb, edge_weight, edge_index, users, items):
  e_total = edge_weight.shape[0]
  per_w = -(-e_total // (NW * 4 * CH)) * 4 * CH
  nchunks = per_w // CH
  e_pad = NW * per_w

  emb0 = jnp.concatenate([user_emb, item_emb], axis=0)
  emb0 = jnp.pad(emb0, ((0, NP - NNODES), (0, 0)))

  src = jnp.pad(edge_index[0], (0, e_pad - e_total))
  dst = jnp.pad(edge_index[1], (0, e_pad - e_total))
  w = jnp.pad(edge_weight, (0, e_pad - e_total))
  epk = jnp.concatenate([
      src.reshape(-1, NG, GRP),
      dst.reshape(-1, NG, GRP),
      lax.bitcast_convert_type(w, jnp.int32).reshape(-1, NG, GRP),
  ], axis=1)
  zeros = jnp.zeros((RPS, D), jnp.float32)

  part1 = _make_scatter1(nchunks)(emb0, epk, zeros)
  mscat = _make_merge_scatter(nchunks)
  part2, emb1, tot1 = mscat(emb0, emb0, part1, epk, zeros)
  part3, emb2, tot2 = mscat(emb1, tot1, part2, epk, zeros)
  gamma, _ = _make_readout()(emb2, tot2, part3, users, items + NUSERS)
  return gamma
